# baseline (speedup 1.0000x reference)
import functools
import jax
import jax.numpy as jnp
from jax import lax
from jax.experimental import pallas as pl
from jax.experimental.pallas import tpu as pltpu
from jax.experimental.pallas import tpu_sc as plsc

_N = 10000
_E = 320000
_D = 128
_G = 4
_OUT = 128
_CH = _D // _G
_NS = 16
_EPS = _E // _NS
_B = 1000
_CNK = _EPS // _B
_ZR = 2 * _N // _NS
_SPLITS = [(j * 128, 128) for j in range(7)] + [(896, 104)]
_NV = _B // 16


def _sc_aggregate(x2, adj, zeros_hbm):
    mesh = plsc.VectorSubcoreMesh(core_axis_name="c", subcore_axis_name="s")

    @functools.partial(
        pl.kernel,
        out_type=jax.ShapeDtypeStruct((_N, _D), jnp.float32),
        mesh=mesh,
        scratch_types=[
            pltpu.VMEM((4, _B), jnp.int32),
            pltpu.VMEM((4, _B), jnp.int32),
            pltpu.VMEM((2, _B), jnp.int32),
            pltpu.VMEM((2, _B, _CH), jnp.float32),
            pltpu.VMEM_SHARED((2, _N, _CH), jnp.float32),
            pltpu.SemaphoreType.DMA,
            pltpu.SemaphoreType.DMA,
            pltpu.SemaphoreType.DMA,
        ],
        compiler_params=pltpu.CompilerParams(use_tc_tiling_on_sc=False),
    )
    def k(x_hbm, adj_hbm, z_hbm, agg_hbm, idx_r, idx_c, idx_g, data_v, acc,
          sem_g0, sem_g1, sem_i):
        c = lax.axis_index("c")
        s = lax.axis_index("s")

        gz = s // 8
        oz = (s % 8) * _ZR
        pltpu.sync_copy(z_hbm, acc.at[gz, pl.ds(oz, _ZR)])
        plsc.subcore_barrier()

        TT = 2 * _CNK

        def fire_idx(t):
            tw = t % TT
            slot = t % 4
            gl = tw // _CNK
            off = s * _EPS + (tw % _CNK) * _B
            pltpu.async_copy(adj_hbm.at[2 * (2 * c + gl), pl.ds(off, _B)],
                             idx_r.at[slot], sem_i)
            pltpu.async_copy(adj_hbm.at[2 * (2 * c + gl) + 1, pl.ds(off, _B)],
                             idx_c.at[slot], sem_i)

        def drain_idx(slot):
            pltpu.make_async_copy(adj_hbm.at[0, pl.ds(0, _B)],
                                  idx_r.at[slot], sem_i).wait()
            pltpu.make_async_copy(adj_hbm.at[0, pl.ds(0, _B)],
                                  idx_c.at[slot], sem_i).wait()

        def transform(t):
            slot = t % 4
            gslot = t % 2
            g = 2 * c + (t % TT) // _CNK
            for v in range(_NV):
                idx_g[gslot, pl.ds(16 * v, 16)] = (
                    idx_c[slot, pl.ds(16 * v, 16)] * _G + g)
            idx_g[gslot, pl.ds(_B - 16, 16)] = (
                idx_c[slot, pl.ds(_B - 16, 16)] * _G + g)

        def fire_gathers(t, sem):
            p = t % 2
            return [
                pltpu.async_copy(
                    x_hbm.at[idx_g.at[p, pl.ds(o, n)]],
                    data_v.at[p, pl.ds(o, n)],
                    sem,
                )
                for o, n in _SPLITS
            ]

        def drain_gathers(t, sem):
            p = t % 2
            for o, n in _SPLITS:
                pltpu.make_async_copy(
                    x_hbm.at[idx_g.at[p, pl.ds(o, n)]],
                    data_v.at[p, pl.ds(o, n)],
                    sem,
                ).wait()

        def fire_scatters(t, sem):
            p = t % 2
            slot = t % 4
            gl = t // _CNK
            return [
                pltpu.async_copy(
                    data_v.at[p, pl.ds(o, n)],
                    acc.at[gl].at[idx_r.at[slot, pl.ds(o, n)]],
                    sem,
                    add=True,
                )
                for o, n in _SPLITS
            ]

        fire_idx(0)
        fire_idx(1)
        drain_idx(0)
        transform(0)
        fire_gathers(0, sem_g0)
        fire_idx(2)
        drain_idx(1)
        transform(1)
        fire_gathers(1, sem_g1)
        fire_idx(3)

        def body(i, carry):
            a = 2 * i
            b = a + 1
            drain_gathers(a, sem_g0)
            sca = fire_scatters(a, sem_g0)
            drain_gathers(b, sem_g1)
            scb = fire_scatters(b, sem_g1)
            drain_idx((a + 2) % 4)
            transform(a + 2)
            for d in sca:
                d.wait()
            fire_gathers(a + 2, sem_g0)
            drain_idx((b + 2) % 4)
            transform(b + 2)
            for d in scb:
                d.wait()
            fire_gathers(b + 2, sem_g1)
            fire_idx(a + 4)
            fire_idx(b + 4)
            return carry

        lax.fori_loop(0, _CNK, body, 0)

        drain_gathers(0, sem_g0)
        drain_gathers(1, sem_g1)
        drain_idx(2)
        drain_idx(3)

        plsc.subcore_barrier()

        pltpu.sync_copy(
            acc.at[gz, pl.ds(oz, _ZR)],
            agg_hbm.at[pl.ds(oz, _ZR), pl.ds((2 * c + gz) * _CH, _CH)],
        )

    return k(x2, adj, zeros_hbm)


_BN = 1000


def _tc_matmul_body(agg_ref, w_ref, b_ref, out_ref):
    out_ref[...] = (
        jnp.dot(agg_ref[...], w_ref[...], preferred_element_type=jnp.float32)
        + b_ref[...]
    )


def _tc_matmul(agg, W, b):
    return pl.pallas_call(
        _tc_matmul_body,
        grid=(_N // _BN,),
        in_specs=[
            pl.BlockSpec((_BN, _D), lambda i: (i, 0)),
            pl.BlockSpec((_D, _OUT), lambda i: (0, 0)),
            pl.BlockSpec((1, _OUT), lambda i: (0, 0)),
        ],
        out_specs=pl.BlockSpec((_BN, _OUT), lambda i: (i, 0)),
        out_shape=jax.ShapeDtypeStruct((_N, _OUT), jnp.float32),
    )(agg, W, b.reshape(1, _OUT))


@jax.jit
def kernel(input, adj, W, b):
    x2 = input.reshape(_N * _G, _CH)
    adj8 = adj.reshape(2 * _G, _E)
    zeros_hbm = jnp.zeros((_ZR, _CH), jnp.float32)
    agg = _sc_aggregate(x2, adj8, zeros_hbm)
    return _tc_matmul(agg, W, b)


# speedup vs baseline: 1.0586x; 1.0586x over previous
import functools
import jax
import jax.numpy as jnp
from jax import lax
from jax.experimental import pallas as pl
from jax.experimental.pallas import tpu as pltpu
from jax.experimental.pallas import tpu_sc as plsc

_N = 10000
_E = 320000
_D = 128
_G = 4
_OUT = 128
_CH = _D // _G
_NS = 16
_EPS = _E // _NS
_B = 2000
_CNK = _EPS // _B
_ZR = 2 * _N // _NS
_SPLITS = [(j * 128, 128) for j in range(15)] + [(1920, 80)]
_NV = _B // 16


def _sc_aggregate(x2, adj, zeros_hbm):
    mesh = plsc.VectorSubcoreMesh(core_axis_name="c", subcore_axis_name="s")

    @functools.partial(
        pl.kernel,
        out_type=jax.ShapeDtypeStruct((_N, _D), jnp.float32),
        mesh=mesh,
        scratch_types=[
            pltpu.VMEM((2, _B), jnp.int32),
            pltpu.VMEM((2, _B), jnp.int32),
            pltpu.VMEM((2, _B), jnp.int32),
            pltpu.VMEM((_B, _CH), jnp.float32),
            pltpu.VMEM_SHARED((2, _N, _CH), jnp.float32),
            pltpu.SemaphoreType.DMA,
            pltpu.SemaphoreType.DMA,
        ],
        compiler_params=pltpu.CompilerParams(use_tc_tiling_on_sc=False),
    )
    def k(x_hbm, adj_hbm, z_hbm, agg_hbm, idx_r, idx_c, idx_g, data_v, acc,
          sem_g, sem_i):
        c = lax.axis_index("c")
        s = lax.axis_index("s")

        gz = s // 8
        oz = (s % 8) * _ZR
        pltpu.sync_copy(z_hbm, acc.at[gz, pl.ds(oz, _ZR)])
        plsc.subcore_barrier()

        TT = 2 * _CNK

        def fire_idx(t, slot):
            tw = t % TT
            gl = tw // _CNK
            off = s * _EPS + (tw % _CNK) * _B
            pltpu.async_copy(adj_hbm.at[2 * (2 * c + gl), pl.ds(off, _B)],
                             idx_r.at[slot], sem_i)
            pltpu.async_copy(adj_hbm.at[2 * (2 * c + gl) + 1, pl.ds(off, _B)],
                             idx_c.at[slot], sem_i)

        def drain_idx(slot):
            pltpu.make_async_copy(adj_hbm.at[0, pl.ds(0, _B)],
                                  idx_r.at[slot], sem_i).wait()
            pltpu.make_async_copy(adj_hbm.at[0, pl.ds(0, _B)],
                                  idx_c.at[slot], sem_i).wait()

        def transform(t, slot):
            g = 2 * c + (t % TT) // _CNK
            for v in range(_NV):
                idx_g[slot, pl.ds(16 * v, 16)] = (
                    idx_c[slot, pl.ds(16 * v, 16)] * _G + g)

        fire_idx(0, 0)
        drain_idx(0)
        transform(0, 0)
        fire_idx(1, 1)

        def chunk(t, carry):
            p = t % 2
            gl = t // _CNK
            descs = [
                pltpu.async_copy(
                    x_hbm.at[idx_g.at[p, pl.ds(o, n)]],
                    data_v.at[pl.ds(o, n)],
                    sem_g,
                )
                for o, n in _SPLITS
            ]
            drain_idx(1 - p)
            transform(t + 1, 1 - p)
            for d, (o, n) in zip(descs, _SPLITS):
                d.wait()
                pltpu.sync_copy(
                    data_v.at[pl.ds(o, n)],
                    acc.at[gl].at[idx_r.at[p, pl.ds(o, n)]],
                    add=True,
                )
            fire_idx(t + 2, p)
            return carry

        lax.fori_loop(0, TT, chunk, 0)
        drain_idx(1)

        plsc.subcore_barrier()

        pltpu.sync_copy(
            acc.at[gz, pl.ds(oz, _ZR)],
            agg_hbm.at[pl.ds(oz, _ZR), pl.ds((2 * c + gz) * _CH, _CH)],
        )

    return k(x2, adj, zeros_hbm)


_BN = 1000


def _tc_matmul_body(agg_ref, w_ref, b_ref, out_ref):
    out_ref[...] = (
        jnp.dot(agg_ref[...], w_ref[...], preferred_element_type=jnp.float32)
        + b_ref[...]
    )


def _tc_matmul(agg, W, b):
    return pl.pallas_call(
        _tc_matmul_body,
        grid=(_N // _BN,),
        in_specs=[
            pl.BlockSpec((_BN, _D), lambda i: (i, 0)),
            pl.BlockSpec((_D, _OUT), lambda i: (0, 0)),
            pl.BlockSpec((1, _OUT), lambda i: (0, 0)),
        ],
        out_specs=pl.BlockSpec((_BN, _OUT), lambda i: (i, 0)),
        out_shape=jax.ShapeDtypeStruct((_N, _OUT), jnp.float32),
    )(agg, W, b.reshape(1, _OUT))


@jax.jit
def kernel(input, adj, W, b):
    x2 = input.reshape(_N * _G, _CH)
    adj8 = adj.reshape(2 * _G, _E)
    zeros_hbm = jnp.zeros((_ZR, _CH), jnp.float32)
    agg = _sc_aggregate(x2, adj8, zeros_hbm)
    return _tc_matmul(agg, W, b)


# speedup vs baseline: 1.0651x; 1.0061x over previous
import functools
import jax
import jax.numpy as jnp
from jax import lax
from jax.experimental import pallas as pl
from jax.experimental.pallas import tpu as pltpu
from jax.experimental.pallas import tpu_sc as plsc

_N = 10000
_E = 320000
_D = 128
_G = 4
_OUT = 128
_CH = _D // _G
_NS = 16
_EPS = _E // _NS
_B = 2000
_CNK = _EPS // _B
_ZR = 2 * _N // _NS
_SPLITS = [(j * 128, 128) for j in range(15)] + [(1920, 80)]
_NV = _B // 16


def _sc_aggregate(x2, adj, zeros_hbm):
    mesh = plsc.VectorSubcoreMesh(core_axis_name="c", subcore_axis_name="s")

    @functools.partial(
        pl.kernel,
        out_type=jax.ShapeDtypeStruct((_N, _D), jnp.float32),
        mesh=mesh,
        scratch_types=[
            pltpu.VMEM((2, _B), jnp.int32),
            pltpu.VMEM((2, _B), jnp.int32),
            pltpu.VMEM((2, _B), jnp.int32),
            pltpu.VMEM((_B, _CH), jnp.float32),
            pltpu.VMEM_SHARED((2, _N, _CH), jnp.float32),
            pltpu.SemaphoreType.DMA,
            pltpu.SemaphoreType.DMA,
        ],
        compiler_params=pltpu.CompilerParams(use_tc_tiling_on_sc=False),
    )
    def k(x_hbm, adj_hbm, z_hbm, agg_hbm, idx_r, idx_c, idx_g, data_v, acc,
          sem_g, sem_i):
        c = lax.axis_index("c")
        s = lax.axis_index("s")

        gz = s // 8
        oz = (s % 8) * _ZR
        pltpu.sync_copy(z_hbm, acc.at[gz, pl.ds(oz, _ZR)])
        plsc.subcore_barrier()

        TT = 2 * _CNK

        def fire_idx(t, slot):
            tw = t % TT
            gl = tw // _CNK
            off = s * _EPS + (tw % _CNK) * _B
            pltpu.async_copy(adj_hbm.at[2 * (2 * c + gl), pl.ds(off, _B)],
                             idx_r.at[slot], sem_i)
            pltpu.async_copy(adj_hbm.at[2 * (2 * c + gl) + 1, pl.ds(off, _B)],
                             idx_c.at[slot], sem_i)

        def drain_idx(slot):
            pltpu.make_async_copy(adj_hbm.at[0, pl.ds(0, _B)],
                                  idx_r.at[slot], sem_i).wait()
            pltpu.make_async_copy(adj_hbm.at[0, pl.ds(0, _B)],
                                  idx_c.at[slot], sem_i).wait()

        def transform(t, slot):
            g = 2 * c + (t % TT) // _CNK
            for v in range(_NV):
                idx_g[slot, pl.ds(16 * v, 16)] = (
                    idx_c[slot, pl.ds(16 * v, 16)] * _G + g)

        fire_idx(0, 0)
        drain_idx(0)
        transform(0, 0)
        fire_idx(1, 1)

        def chunk(t, carry):
            p = t % 2
            gl = t // _CNK
            descs = [
                pltpu.async_copy(
                    x_hbm.at[idx_g.at[p, pl.ds(o, n)]],
                    data_v.at[pl.ds(o, n)],
                    sem_g,
                )
                for o, n in _SPLITS
            ]
            drain_idx(1 - p)
            transform(t + 1, 1 - p)
            for d, (o, n) in zip(descs, _SPLITS):
                d.wait()
                pltpu.sync_copy(
                    data_v.at[pl.ds(o, n)],
                    acc.at[gl].at[idx_r.at[p, pl.ds(o, n)]],
                    add=True,
                )
            fire_idx(t + 2, p)
            return carry

        lax.fori_loop(0, TT, chunk, 0)
        drain_idx(1)

        plsc.subcore_barrier()

        pltpu.sync_copy(
            acc.at[gz, pl.ds(oz, _ZR)],
            agg_hbm.at[pl.ds(oz, _ZR), pl.ds((2 * c + gz) * _CH, _CH)],
        )

    return k(x2, adj, zeros_hbm)


_EB = 32000


def _adj_prep_body(adj_ref, out_ref):
    out_ref[...] = adj_ref[...].reshape(2 * _G, _EB)


def _adj_prep(adj):
    return pl.pallas_call(
        _adj_prep_body,
        grid=(_E // _EB,),
        in_specs=[pl.BlockSpec((_G, 2, _EB), lambda i: (0, 0, i))],
        out_specs=pl.BlockSpec((2 * _G, _EB), lambda i: (0, i)),
        out_shape=jax.ShapeDtypeStruct((2 * _G, _E), jnp.int32),
    )(adj)


_BN = 1000


def _tc_matmul_body(agg_ref, w_ref, b_ref, out_ref):
    out_ref[...] = (
        jnp.dot(agg_ref[...], w_ref[...], preferred_element_type=jnp.float32)
        + b_ref[...]
    )


def _tc_matmul(agg, W, b):
    return pl.pallas_call(
        _tc_matmul_body,
        grid=(_N // _BN,),
        in_specs=[
            pl.BlockSpec((_BN, _D), lambda i: (i, 0)),
            pl.BlockSpec((_D, _OUT), lambda i: (0, 0)),
            pl.BlockSpec((1, _OUT), lambda i: (0, 0)),
        ],
        out_specs=pl.BlockSpec((_BN, _OUT), lambda i: (i, 0)),
        out_shape=jax.ShapeDtypeStruct((_N, _OUT), jnp.float32),
    )(agg, W, b.reshape(1, _OUT))


@jax.jit
def kernel(input, adj, W, b):
    x2 = input.reshape(_N * _G, _CH)
    adj8 = _adj_prep(adj)
    zeros_hbm = jnp.zeros((_ZR, _CH), jnp.float32)
    agg = _sc_aggregate(x2, adj8, zeros_hbm)
    return _tc_matmul(agg, W, b)
